# Initial kernel scaffold; baseline (speedup 1.0000x reference)
import functools

import jax
import jax.numpy as jnp
from jax import lax
from jax.experimental import pallas as pl
from jax.experimental.pallas import tpu as pltpu

MARGIN = 1.1
WEIGHT = 1.0
EPS = 1e-6
LOWE_RATIO_TH = 0.9

N = 2048
D = 512
BN = 256
NBLK = N // BN


def _norm_rows(x):
    n = jnp.sqrt(jnp.sum(x * x, axis=-1, keepdims=True))
    return x / jnp.maximum(n, 1e-12)


def _fused_kernel(nimg, q_ref, sf_ref, out_ref,
                  qn_s, a2_s, colv_s, coli1_s, coli2_s,
                  rowv_s, rowi_s, y_s):
    j = pl.program_id(0)

    @pl.when(j == 0)
    def _init():
        qn = _norm_rows(q_ref[...])
        qn_s[...] = qn
        a2_s[...] = jnp.sum(qn * qn, axis=1, keepdims=True)
        rowv_s[...] = jnp.full((N, 1), jnp.inf, jnp.float32)
        rowi_s[...] = jnp.zeros((N, 1), jnp.int32)

    sfb = sf_ref[...]
    sfn = _norm_rows(sfb)
    pn = sfn[1]
    b2 = jnp.sum(pn * pn, axis=1)

    qn = qn_s[...]
    g = jnp.dot(qn, pn.T, preferred_element_type=jnp.float32)
    d2 = jnp.maximum(a2_s[...] + b2[None, :] - 2.0 * g, 0.0)

    v1 = jnp.min(d2, axis=0)
    i1 = jnp.argmin(d2, axis=0).astype(jnp.int32)
    riota = lax.broadcasted_iota(jnp.int32, d2.shape, 0)
    masked = jnp.where(riota == i1[None, :], jnp.inf, d2)
    i2 = jnp.argmin(masked, axis=0).astype(jnp.int32)
    colv_s[j, :] = v1
    coli1_s[j, :] = i1
    coli2_s[j, :] = i2

    rv = jnp.min(d2, axis=1, keepdims=True)
    ri = jnp.argmin(d2, axis=1).astype(jnp.int32)[:, None] + j * BN
    take = rv < rowv_s[...]
    rowv_s[...] = jnp.where(take, rv, rowv_s[...])
    rowi_s[...] = jnp.where(take, ri, rowi_s[...])

    qb = sfn[0]
    dif = qb - pn + EPS
    dm = jnp.sqrt(jnp.sum(dif * dif, axis=1))
    y = dm * dm
    for k in range(2, nimg):
        dif = qb - sfn[k] + EPS
        dm = jnp.sqrt(jnp.sum(dif * dif, axis=1))
        h = jnp.maximum(MARGIN - dm, 0.0)
        y = y + h * h
    y_s[j, :] = y

    @pl.when(j == NBLK - 1)
    def _final():
        gidx = (lax.broadcasted_iota(jnp.int32, (NBLK, BN), 0) * BN
                + lax.broadcasted_iota(jnp.int32, (NBLK, BN), 1))
        rowi = rowi_s[...].reshape(NBLK, BN)
        dbest = jnp.sqrt(colv_s[...])
        ratio = dbest / coli2_s[...].astype(jnp.float32)
        valid = jnp.logical_and(
            jnp.logical_and(coli1_s[...] == gidx, rowi == gidx),
            ratio <= LOWE_RATIO_TH)
        total = 0.5 * jnp.sum(jnp.where(valid, y_s[...], 0.0))
        anyv = jnp.any(valid)
        out_ref[0, 0] = jnp.where(anyv, total * WEIGHT, jnp.float32(0.0))


@jax.jit
def kernel(superfeatures, target):
    del target
    nimg = superfeatures.shape[0]
    q = superfeatures[0]
    out = pl.pallas_call(
        functools.partial(_fused_kernel, nimg),
        grid=(NBLK,),
        in_specs=[
            pl.BlockSpec((N, D), lambda j: (0, 0)),
            pl.BlockSpec((nimg, BN, D), lambda j: (0, j, 0)),
        ],
        out_specs=pl.BlockSpec((1, 1), lambda j: (0, 0)),
        out_shape=jax.ShapeDtypeStruct((1, 1), jnp.float32),
        scratch_shapes=[
            pltpu.VMEM((N, D), jnp.float32),
            pltpu.VMEM((N, 1), jnp.float32),
            pltpu.VMEM((NBLK, BN), jnp.float32),
            pltpu.VMEM((NBLK, BN), jnp.int32),
            pltpu.VMEM((NBLK, BN), jnp.int32),
            pltpu.VMEM((N, 1), jnp.float32),
            pltpu.VMEM((N, 1), jnp.int32),
            pltpu.VMEM((NBLK, BN), jnp.float32),
        ],
    )(q, superfeatures)
    return out[0, 0]


# speedup vs baseline: 6.9774x; 6.9774x over previous
import functools

import jax
import jax.numpy as jnp
from jax import lax
from jax.experimental import pallas as pl
from jax.experimental.pallas import tpu as pltpu

MARGIN = 1.1
WEIGHT = 1.0
EPS = 1e-6
LOWE_RATIO_TH = 0.9

N = 2048
D = 512
BN = 256
NBLK = N // BN


def _norm_rows(x):
    n = jnp.sqrt(jnp.sum(x * x, axis=-1, keepdims=True))
    return x / jnp.maximum(n, 1e-12)


def _fused_kernel(nimg, q_ref, sf_ref, out_ref,
                  qn_s, a2_s, colv_s, coli1_s, coli2_s,
                  rowv_s, rowi_s, y_s):
    j = pl.program_id(0)

    @pl.when(j == 0)
    def _init():
        qn = _norm_rows(q_ref[...])
        qn_s[...] = qn
        a2_s[...] = jnp.sum(qn * qn, axis=1, keepdims=True)
        rowv_s[...] = jnp.full((N, 1), jnp.inf, jnp.float32)
        rowi_s[...] = jnp.zeros((N, 1), jnp.int32)

    sfb = sf_ref[...]
    sfn = _norm_rows(sfb)
    pn = sfn[1]
    b2 = jnp.sum(pn * pn, axis=1)

    qn = qn_s[...]
    g = jnp.dot(qn, pn.T, preferred_element_type=jnp.float32)
    d2 = jnp.maximum(a2_s[...] + b2[None, :] - 2.0 * g, 0.0)

    v1 = jnp.min(d2, axis=0)
    i1 = jnp.argmin(d2, axis=0).astype(jnp.int32)
    riota = lax.broadcasted_iota(jnp.int32, d2.shape, 0)
    masked = jnp.where(riota == i1[None, :], jnp.inf, d2)
    i2 = jnp.argmin(masked, axis=0).astype(jnp.int32)
    colv_s[j, :] = v1
    coli1_s[j, :] = i1
    coli2_s[j, :] = i2

    rv = jnp.min(d2, axis=1, keepdims=True)
    ri = jnp.argmin(d2, axis=1).astype(jnp.int32)[:, None] + j * BN
    take = rv < rowv_s[...]
    rowv_s[...] = jnp.where(take, rv, rowv_s[...])
    rowi_s[...] = jnp.where(take, ri, rowi_s[...])

    qb = sfn[0]
    dif = qb - pn + EPS
    dm = jnp.sqrt(jnp.sum(dif * dif, axis=1))
    y = dm * dm
    for k in range(2, nimg):
        dif = qb - sfn[k] + EPS
        dm = jnp.sqrt(jnp.sum(dif * dif, axis=1))
        h = jnp.maximum(MARGIN - dm, 0.0)
        y = y + h * h
    y_s[j, :] = y

    @pl.when(j == NBLK - 1)
    def _final():
        gidx = (lax.broadcasted_iota(jnp.int32, (NBLK, BN), 0) * BN
                + lax.broadcasted_iota(jnp.int32, (NBLK, BN), 1))
        rowi = rowi_s[...].reshape(NBLK, BN)
        dbest = jnp.sqrt(colv_s[...])
        ratio = dbest / coli2_s[...].astype(jnp.float32)
        valid = jnp.logical_and(
            jnp.logical_and(coli1_s[...] == gidx, rowi == gidx),
            ratio <= LOWE_RATIO_TH)
        total = 0.5 * jnp.sum(jnp.where(valid, y_s[...], 0.0))
        anyv = jnp.any(valid)
        res = jnp.where(anyv, total * WEIGHT, jnp.float32(0.0))
        out_ref[...] = jnp.broadcast_to(res, (1, 1))


@jax.jit
def kernel(superfeatures, target):
    del target
    nimg = superfeatures.shape[0]
    q = superfeatures[0]
    out = pl.pallas_call(
        functools.partial(_fused_kernel, nimg),
        grid=(NBLK,),
        in_specs=[
            pl.BlockSpec((N, D), lambda j: (0, 0)),
            pl.BlockSpec((nimg, BN, D), lambda j: (0, j, 0)),
        ],
        out_specs=pl.BlockSpec((1, 1), lambda j: (0, 0)),
        out_shape=jax.ShapeDtypeStruct((1, 1), jnp.float32),
        scratch_shapes=[
            pltpu.VMEM((N, D), jnp.float32),
            pltpu.VMEM((N, 1), jnp.float32),
            pltpu.VMEM((NBLK, BN), jnp.float32),
            pltpu.VMEM((NBLK, BN), jnp.int32),
            pltpu.VMEM((NBLK, BN), jnp.int32),
            pltpu.VMEM((N, 1), jnp.float32),
            pltpu.VMEM((N, 1), jnp.int32),
            pltpu.VMEM((NBLK, BN), jnp.float32),
        ],
    )(q, superfeatures)
    return out[0, 0]
